# Initial kernel scaffold; baseline (speedup 1.0000x reference)
import jax
import jax.numpy as jnp
from jax.experimental import pallas as pl


def kernel(x, edge_index, W_src1, W_dst1, att_src1, att_dst1, b1, W_src2, W_dst2, att_src2, att_dst2, b2):
    raise NotImplementedError("write your pallas kernel here")


# speedup vs baseline: 19.2225x; 19.2225x over previous
import functools

import jax
import jax.numpy as jnp
from jax import lax
from jax.experimental import pallas as pl
from jax.experimental.pallas import tpu as pltpu
from jax.experimental.pallas import tpu_sc as plsc

N = 10000
E = 320000
D = 128
NC = 2
NS = 16
NW = NC * NS
EW = E // NW
K = 80
CH = EW // K
ZR = 16
NCK = N // ZR

_mesh = plsc.VectorSubcoreMesh(core_axis_name="c", subcore_axis_name="s")


@functools.partial(
    pl.kernel,
    mesh=_mesh,
    compiler_params=pltpu.CompilerParams(needs_layout_passes=False),
    out_type=[
        jax.ShapeDtypeStruct((NC, N, D), jnp.float32),
        jax.ShapeDtypeStruct((NW, N), jnp.float32),
    ],
    scratch_types=[
        pltpu.VMEM_SHARED((N, D), jnp.float32),
        pltpu.VMEM((N,), jnp.float32),
        pltpu.VMEM((N,), jnp.float32),
        pltpu.VMEM((N,), jnp.float32),
        pltpu.VMEM((K,), jnp.int32),
        pltpu.VMEM((K,), jnp.int32),
        pltpu.VMEM((K,), jnp.float32),
        pltpu.VMEM((K, D), jnp.float32),
        pltpu.VMEM((ZR, D), jnp.float32),
        pltpu.SemaphoreType.DMA,
    ],
)
def _edge_pass(asrc_hbm, adst_hbm, xs_hbm, src_hbm, dst_hbm, acc_out, den_out,
               acc_sh, asrc_t, adst_t, den_t, src_v, dst_v, ev_v, rows_v,
               zbuf, sem):
    cid = lax.axis_index("c")
    sid = lax.axis_index("s")
    wid = sid * NC + cid
    z16 = jnp.zeros((16,), jnp.float32)

    def zb(i, carry):
        for c in range(D // 16):
            zbuf[i, pl.ds(c * 16, 16)] = z16
        return carry

    lax.fori_loop(0, ZR, zb, 0)

    def zd(i, carry):
        den_t[pl.ds(i * 16, 16)] = z16
        return carry

    lax.fori_loop(0, N // 16, zd, 0)

    def zs(j, carry):
        ckid = sid + j * NS

        @pl.when(ckid < NCK)
        def _():
            off = pl.multiple_of(ckid * ZR, ZR)
            pltpu.sync_copy(zbuf, acc_sh.at[pl.ds(off, ZR)])

        return carry

    lax.fori_loop(0, (NCK + NS - 1) // NS, zs, 0)

    pltpu.sync_copy(asrc_hbm, asrc_t)
    pltpu.sync_copy(adst_hbm, adst_t)
    plsc.subcore_barrier()

    lanes = lax.iota(jnp.int32, 16)
    masks = [lanes == l for l in range(16)]

    def chunk(ch, carry):
        base = wid * EW + ch * K
        pltpu.sync_copy(src_hbm.at[pl.ds(base, K)], src_v)
        pltpu.sync_copy(dst_hbm.at[pl.ds(base, K)], dst_v)
        cp = pltpu.async_copy(xs_hbm.at[src_v], rows_v, sem)
        for j in range(K // 16):
            si = src_v[pl.ds(j * 16, 16)]
            di = dst_v[pl.ds(j * 16, 16)]
            a = plsc.load_gather(asrc_t, [si]) + plsc.load_gather(adst_t, [di])
            a = jnp.where(a >= 0.0, a, a * 0.2)
            e = jnp.exp(a)
            ev_v[pl.ds(j * 16, 16)] = e
            for l in range(16):
                plsc.addupdate_scatter(den_t, [di], e, mask=masks[l])
        cp.wait()

        def row_group(g, rcarry):
            ev16 = ev_v[pl.ds(g * 16, 16)]
            for l in range(16):
                r = g * 16 + l
                sv = jnp.full((16,), ev16[l], jnp.float32)
                for c in range(D // 16):
                    rows_v[r, pl.ds(c * 16, 16)] = (
                        rows_v[r, pl.ds(c * 16, 16)] * sv)
            return rcarry

        lax.fori_loop(0, K // 16, row_group, 0)
        pltpu.sync_copy(rows_v, acc_sh.at[dst_v], add=True)
        return carry

    lax.fori_loop(0, CH, chunk, 0)
    plsc.subcore_barrier()

    def wb(j, carry):
        ckid = sid + j * NS

        @pl.when(ckid < NCK)
        def _():
            off = pl.multiple_of(ckid * ZR, ZR)
            pltpu.sync_copy(acc_sh.at[pl.ds(off, ZR)],
                            acc_out.at[cid, pl.ds(off, ZR)])

        return carry

    lax.fori_loop(0, (NCK + NS - 1) // NS, wb, 0)
    pltpu.sync_copy(den_t, den_out.at[wid])


_BN = 1000


def _tc1_body(x_ref, ws_ref, as_ref, wd_ref, ad_ref, xs_ref, asrc_ref, adst_ref):
    xs = jnp.dot(x_ref[...], ws_ref[...], preferred_element_type=jnp.float32)
    xs_ref[...] = xs
    asrc_ref[...] = jnp.dot(xs, as_ref[...], preferred_element_type=jnp.float32)
    u = jnp.dot(wd_ref[...], ad_ref[...], preferred_element_type=jnp.float32)
    adst_ref[...] = jnp.dot(x_ref[...], u, preferred_element_type=jnp.float32)


def _tc1(x, ws, a_s, wd, a_d):
    return pl.pallas_call(
        _tc1_body,
        grid=(N // _BN,),
        in_specs=[
            pl.BlockSpec((_BN, D), lambda i: (i, 0)),
            pl.BlockSpec((D, D), lambda i: (0, 0)),
            pl.BlockSpec((D, 1), lambda i: (0, 0)),
            pl.BlockSpec((D, D), lambda i: (0, 0)),
            pl.BlockSpec((D, 1), lambda i: (0, 0)),
        ],
        out_specs=[
            pl.BlockSpec((_BN, D), lambda i: (i, 0)),
            pl.BlockSpec((_BN, 1), lambda i: (i, 0)),
            pl.BlockSpec((_BN, 1), lambda i: (i, 0)),
        ],
        out_shape=[
            jax.ShapeDtypeStruct((N, D), jnp.float32),
            jax.ShapeDtypeStruct((N, 1), jnp.float32),
            jax.ShapeDtypeStruct((N, 1), jnp.float32),
        ],
    )(x, ws, a_s, wd, a_d)


def _combine(acc_ref, den_ref, b_ref):
    s = acc_ref[0] + acc_ref[1]
    den = jnp.sum(den_ref[...], axis=0)
    return s / (den + 1e-16) + b_ref[...]


def _tc2_body(acc_ref, den_ref, b1_ref, ws_ref, as_ref, wd_ref, ad_ref,
              x1_ref, xs2_ref, asrc_ref, adst_ref):
    x1 = jnp.maximum(_combine(acc_ref, den_ref, b1_ref), 0.0)
    x1_ref[...] = x1
    xs2 = jnp.dot(x1, ws_ref[...], preferred_element_type=jnp.float32)
    xs2_ref[...] = xs2
    asrc_ref[...] = jnp.dot(xs2, as_ref[...], preferred_element_type=jnp.float32)
    u = jnp.dot(wd_ref[...], ad_ref[...], preferred_element_type=jnp.float32)
    adst_ref[...] = jnp.dot(x1, u, preferred_element_type=jnp.float32)


def _tc2(acc, den, b1, ws, a_s, wd, a_d):
    return pl.pallas_call(
        _tc2_body,
        grid=(N // _BN,),
        in_specs=[
            pl.BlockSpec((NC, _BN, D), lambda i: (0, i, 0)),
            pl.BlockSpec((NW, _BN, 1), lambda i: (0, i, 0)),
            pl.BlockSpec((1, D), lambda i: (0, 0)),
            pl.BlockSpec((D, D), lambda i: (0, 0)),
            pl.BlockSpec((D, 1), lambda i: (0, 0)),
            pl.BlockSpec((D, D), lambda i: (0, 0)),
            pl.BlockSpec((D, 1), lambda i: (0, 0)),
        ],
        out_specs=[
            pl.BlockSpec((_BN, D), lambda i: (i, 0)),
            pl.BlockSpec((_BN, D), lambda i: (i, 0)),
            pl.BlockSpec((_BN, 1), lambda i: (i, 0)),
            pl.BlockSpec((_BN, 1), lambda i: (i, 0)),
        ],
        out_shape=[
            jax.ShapeDtypeStruct((N, D), jnp.float32),
            jax.ShapeDtypeStruct((N, D), jnp.float32),
            jax.ShapeDtypeStruct((N, 1), jnp.float32),
            jax.ShapeDtypeStruct((N, 1), jnp.float32),
        ],
    )(acc, den, b1, ws, a_s, wd, a_d)


def _tc3_body(acc_ref, den_ref, x1_ref, b2_ref, o_ref):
    o_ref[:, :D] = x1_ref[...]
    o_ref[:, D:2 * D] = _combine(acc_ref, den_ref, b2_ref)


def _tc3(acc, den, x1, b2):
    return pl.pallas_call(
        _tc3_body,
        grid=(N // _BN,),
        in_specs=[
            pl.BlockSpec((NC, _BN, D), lambda i: (0, i, 0)),
            pl.BlockSpec((NW, _BN, 1), lambda i: (0, i, 0)),
            pl.BlockSpec((_BN, D), lambda i: (i, 0)),
            pl.BlockSpec((1, D), lambda i: (0, 0)),
        ],
        out_specs=pl.BlockSpec((_BN, 2 * D), lambda i: (i, 0)),
        out_shape=jax.ShapeDtypeStruct((N, 2 * D), jnp.float32),
    )(acc, den, x1, b2)


def kernel(x, edge_index, W_src1, W_dst1, att_src1, att_dst1, b1,
           W_src2, W_dst2, att_src2, att_dst2, b2):
    src = edge_index[0]
    dst = edge_index[1]
    xs1, asrc1, adst1 = _tc1(x, W_src1, att_src1.reshape(D, 1),
                             W_dst1, att_dst1.reshape(D, 1))
    acc1, den1 = _edge_pass(asrc1.reshape(N), adst1.reshape(N), xs1, src, dst)
    den1 = den1.reshape(NW, N, 1)
    x1, xs2, asrc2, adst2 = _tc2(acc1, den1, b1.reshape(1, D), W_src2,
                                 att_src2.reshape(D, 1), W_dst2,
                                 att_dst2.reshape(D, 1))
    acc2, den2 = _edge_pass(asrc2.reshape(N), adst2.reshape(N), xs2, src, dst)
    return _tc3(acc2, den2.reshape(NW, N, 1), x1, b2.reshape(1, D))
